# baseline (speedup 1.0000x reference)
import functools

import jax
import jax.numpy as jnp
from jax import lax
from jax.experimental import pallas as pl
from jax.experimental.pallas import tpu as pltpu
from jax.experimental.pallas import tpu_sc as plsc

EMBED_DIM = 32
LANES = 128
NW = 32


def _sc_gather(x1p, x2p, T1, T2, seq):
  bsz = x1p.shape[0]
  n = bsz * seq
  rpw = bsz // NW
  tpw = rpw * seq
  gpw = tpw // LANES
  mesh = plsc.VectorSubcoreMesh(core_axis_name="c", subcore_axis_name="s")

  @functools.partial(
      pl.kernel,
      out_type=jax.ShapeDtypeStruct((n, LANES), jnp.float32),
      mesh=mesh,
      compiler_params=pltpu.CompilerParams(
          use_tc_tiling_on_sc=False, needs_layout_passes=False),
      scratch_types=(
          pltpu.VMEM((rpw, LANES), jnp.int32),
          pltpu.VMEM((rpw, LANES), jnp.int32),
          pltpu.VMEM((tpw,), jnp.int32),
          pltpu.VMEM((tpw,), jnp.int32),
          pltpu.VMEM((2, LANES, EMBED_DIM), jnp.float32),
          pltpu.VMEM((2, LANES, EMBED_DIM), jnp.float32),
          pltpu.SemaphoreType.DMA,
          pltpu.SemaphoreType.DMA,
          pltpu.SemaphoreType.DMA,
          pltpu.SemaphoreType.DMA,
      ),
  )
  def gather_kernel(x1_hbm, x2_hbm, t1_hbm, t2_hbm, ecat_hbm,
                    idxr1_v, idxr2_v, idxc1_v, idxc2_v, rows1_v, rows2_v,
                    gs1, gs2, ws1, ws2):
    wid = lax.axis_index("s") * 2 + lax.axis_index("c")
    row0 = wid * rpw
    pltpu.sync_copy(x1_hbm.at[pl.ds(row0, rpw)], idxr1_v)
    pltpu.sync_copy(x2_hbm.at[pl.ds(row0, rpw)], idxr2_v)

    def compact(i, carry):
      p = i * 16 + lax.iota(jnp.int32, 16)
      r = p // seq
      c = p % seq
      plsc.store_scatter(idxc1_v, [p], plsc.load_gather(idxr1_v, [r, c]))
      plsc.store_scatter(idxc2_v, [p], plsc.load_gather(idxr2_v, [r, c]))
      return carry

    lax.fori_loop(0, tpw // 16, compact, 0)

    def out_slabs(g):
      tok0 = wid * tpw + g * LANES
      return (ecat_hbm.at[pl.ds(tok0, LANES), pl.ds(0, EMBED_DIM)],
              ecat_hbm.at[pl.ds(tok0, LANES), pl.ds(EMBED_DIM, EMBED_DIM)])

    def body(g, carry):
      b = lax.rem(g, 2)

      @pl.when(g >= 2)
      def _drain_writes():
        o1, o2 = out_slabs(g - 2)
        pltpu.make_async_copy(rows1_v.at[b], o1, ws1).wait()
        pltpu.make_async_copy(rows2_v.at[b], o2, ws2).wait()

      cp1 = pltpu.async_copy(
          t1_hbm.at[idxc1_v.at[pl.ds(g * LANES, LANES)]], rows1_v.at[b], gs1)
      cp2 = pltpu.async_copy(
          t2_hbm.at[idxc2_v.at[pl.ds(g * LANES, LANES)]], rows2_v.at[b], gs2)
      cp1.wait()
      cp2.wait()
      o1, o2 = out_slabs(g)
      pltpu.async_copy(rows1_v.at[b], o1, ws1)
      pltpu.async_copy(rows2_v.at[b], o2, ws2)
      return carry

    lax.fori_loop(0, gpw, body, 0)

    for g_tail in (gpw - 2, gpw - 1):
      b = g_tail % 2
      o1, o2 = out_slabs(g_tail)
      pltpu.make_async_copy(rows1_v.at[b], o1, ws1).wait()
      pltpu.make_async_copy(rows2_v.at[b], o2, ws2).wait()

  return gather_kernel(x1p, x2p, T1, T2)


def _tc_mlp(ecat, w0, b0, w1, b1, w2, b2, wpt, bp):
  n = ecat.shape[0]
  blk = 8192
  grid = n // blk
  d = w1.shape[0]

  def mlp_kernel(ecat_ref, w0_ref, b0_ref, w1_ref, b1_ref,
                 w2_ref, b2_ref, wpt_ref, bp_ref, out_ref):
    e = ecat_ref[:, :d]
    x = jnp.dot(e, w0_ref[...], preferred_element_type=jnp.float32)
    x = jnp.maximum(x + b0_ref[...], 0.0)
    x = jnp.maximum(
        jnp.dot(x, w1_ref[...], preferred_element_type=jnp.float32) + b1_ref[...], 0.0)
    x = jnp.maximum(
        jnp.dot(x, w2_ref[...], preferred_element_type=jnp.float32) + b2_ref[...], 0.0)
    z = jnp.sum(x * wpt_ref[...], axis=1) + bp_ref[0, 0]
    out_ref[...] = jax.nn.sigmoid(z)

  full = lambda shape: pl.BlockSpec(shape, lambda i: (0,) * len(shape))
  return pl.pallas_call(
      mlp_kernel,
      grid=(grid,),
      in_specs=[
          pl.BlockSpec((blk, LANES), lambda i: (i, 0)),
          full((d, d)),
          full((1, d)),
          full((d, d)),
          full((1, d)),
          full((d, d)),
          full((1, d)),
          full((1, d)),
          full((1, 1)),
      ],
      out_specs=pl.BlockSpec((blk,), lambda i: (i,)),
      out_shape=jax.ShapeDtypeStruct((n,), jnp.float32),
  )(ecat, w0, b0, w1, b1, w2, b2, wpt, bp)


def kernel(x1, x2, T1, T2, W0, b0, W1, b1, W2, b2, Wp, bp):
  B, L = x1.shape
  n = B * L
  pad = ((0, 0), (0, LANES - L))
  x1p = jnp.pad(x1.astype(jnp.int32), pad)
  x2p = jnp.pad(x2.astype(jnp.int32), pad)
  ecat = _sc_gather(x1p, x2p, T1, T2, L)
  out = _tc_mlp(
      ecat,
      W0, b0.reshape(1, -1),
      W1, b1.reshape(1, -1), W2, b2.reshape(1, -1),
      Wp.reshape(1, -1), bp.reshape(1, 1))
  return out.reshape(B, L)


# speedup vs baseline: 1.0315x; 1.0315x over previous
import functools

import jax
import jax.numpy as jnp
from jax import lax
from jax.experimental import pallas as pl
from jax.experimental.pallas import tpu as pltpu
from jax.experimental.pallas import tpu_sc as plsc

EMBED_DIM = 32
LANES = 128
NW = 32


def _sc_gather(x1p, x2p, T1, T2, seq):
  bsz = x1p.shape[0]
  n = bsz * seq
  rpw = bsz // NW
  tpw = rpw * seq
  gpw = tpw // LANES
  mesh = plsc.VectorSubcoreMesh(core_axis_name="c", subcore_axis_name="s")

  @functools.partial(
      pl.kernel,
      out_type=jax.ShapeDtypeStruct((n, LANES), jnp.float32),
      mesh=mesh,
      compiler_params=pltpu.CompilerParams(
          use_tc_tiling_on_sc=False, needs_layout_passes=False),
      scratch_types=(
          pltpu.VMEM((rpw, LANES), jnp.int32),
          pltpu.VMEM((rpw, LANES), jnp.int32),
          pltpu.VMEM((tpw,), jnp.int32),
          pltpu.VMEM((tpw,), jnp.int32),
          pltpu.VMEM((2, LANES, EMBED_DIM), jnp.float32),
          pltpu.VMEM((2, LANES, EMBED_DIM), jnp.float32),
          pltpu.SemaphoreType.DMA,
          pltpu.SemaphoreType.DMA,
          pltpu.SemaphoreType.DMA,
          pltpu.SemaphoreType.DMA,
      ),
  )
  def gather_kernel(x1_hbm, x2_hbm, t1_hbm, t2_hbm, ecat_hbm,
                    idxr1_v, idxr2_v, idxc1_v, idxc2_v, rows1_v, rows2_v,
                    gs1, gs2, ws1, ws2):
    wid = lax.axis_index("s") * 2 + lax.axis_index("c")
    row0 = wid * rpw
    pltpu.sync_copy(x1_hbm.at[pl.ds(row0, rpw)], idxr1_v)
    pltpu.sync_copy(x2_hbm.at[pl.ds(row0, rpw)], idxr2_v)

    def compact(i, carry):
      p = i * 16 + lax.iota(jnp.int32, 16)
      r = p // seq
      c = p % seq
      plsc.store_scatter(idxc1_v, [p], plsc.load_gather(idxr1_v, [r, c]))
      plsc.store_scatter(idxc2_v, [p], plsc.load_gather(idxr2_v, [r, c]))
      return carry

    lax.fori_loop(0, tpw // 16, compact, 0)

    def out_slabs(g):
      tok0 = wid * tpw + g * LANES
      return (ecat_hbm.at[pl.ds(tok0, LANES), pl.ds(0, EMBED_DIM)],
              ecat_hbm.at[pl.ds(tok0, LANES), pl.ds(EMBED_DIM, EMBED_DIM)])

    def body(g, carry):
      b = lax.rem(g, 2)

      @pl.when(g >= 2)
      def _drain_writes():
        o1, o2 = out_slabs(g - 2)
        pltpu.make_async_copy(rows1_v.at[b], o1, ws1).wait()
        pltpu.make_async_copy(rows2_v.at[b], o2, ws2).wait()

      cp1 = pltpu.async_copy(
          t1_hbm.at[idxc1_v.at[pl.ds(g * LANES, LANES)]], rows1_v.at[b], gs1)
      cp2 = pltpu.async_copy(
          t2_hbm.at[idxc2_v.at[pl.ds(g * LANES, LANES)]], rows2_v.at[b], gs2)
      cp1.wait()
      cp2.wait()
      o1, o2 = out_slabs(g)
      pltpu.async_copy(rows1_v.at[b], o1, ws1)
      pltpu.async_copy(rows2_v.at[b], o2, ws2)
      return carry

    lax.fori_loop(0, gpw, body, 0)

    for g_tail in (gpw - 2, gpw - 1):
      b = g_tail % 2
      o1, o2 = out_slabs(g_tail)
      pltpu.make_async_copy(rows1_v.at[b], o1, ws1).wait()
      pltpu.make_async_copy(rows2_v.at[b], o2, ws2).wait()

  return gather_kernel(x1p, x2p, T1, T2)


def _tc_mlp(ecat, w0, b0, w1, b1, w2, b2, wpt, bp):
  n = ecat.shape[0]
  blk = 25600
  grid = n // blk
  d = w1.shape[0]

  def mlp_kernel(ecat_ref, w0_ref, b0_ref, w1_ref, b1_ref,
                 w2_ref, b2_ref, wpt_ref, bp_ref, out_ref):
    e = ecat_ref[:, :d]
    x = jnp.dot(e, w0_ref[...], preferred_element_type=jnp.float32)
    x = jnp.maximum(x + b0_ref[...], 0.0)
    x = jnp.maximum(
        jnp.dot(x, w1_ref[...], preferred_element_type=jnp.float32) + b1_ref[...], 0.0)
    x = jnp.maximum(
        jnp.dot(x, w2_ref[...], preferred_element_type=jnp.float32) + b2_ref[...], 0.0)
    z = lax.dot_general(x, wpt_ref[...], (((1,), (1,)), ((), ())),
                        preferred_element_type=jnp.float32)[:, 0] + bp_ref[0, 0]
    out_ref[...] = jax.nn.sigmoid(z)

  full = lambda shape: pl.BlockSpec(shape, lambda i: (0,) * len(shape))
  return pl.pallas_call(
      mlp_kernel,
      grid=(grid,),
      in_specs=[
          pl.BlockSpec((blk, LANES), lambda i: (i, 0)),
          full((d, d)),
          full((1, d)),
          full((d, d)),
          full((1, d)),
          full((d, d)),
          full((1, d)),
          full((1, d)),
          full((1, 1)),
      ],
      out_specs=pl.BlockSpec((blk,), lambda i: (i,)),
      out_shape=jax.ShapeDtypeStruct((n,), jnp.float32),
  )(ecat, w0, b0, w1, b1, w2, b2, wpt, bp)


def kernel(x1, x2, T1, T2, W0, b0, W1, b1, W2, b2, Wp, bp):
  B, L = x1.shape
  n = B * L
  pad = ((0, 0), (0, LANES - L))
  x1p = jnp.pad(x1.astype(jnp.int32), pad)
  x2p = jnp.pad(x2.astype(jnp.int32), pad)
  ecat = _sc_gather(x1p, x2p, T1, T2, L)
  out = _tc_mlp(
      ecat,
      W0, b0.reshape(1, -1),
      W1, b1.reshape(1, -1), W2, b2.reshape(1, -1),
      Wp.reshape(1, -1), bp.reshape(1, 1))
  return out.reshape(B, L)
